# Initial kernel scaffold; baseline (speedup 1.0000x reference)
import functools

import jax
import jax.numpy as jnp
from jax import lax
from jax.experimental import pallas as pl
from jax.experimental.pallas import tpu as pltpu
from jax.experimental.pallas import tpu_sc as plsc

N = 10000
E = 320000
D_IN = 128
D_HID = 128
D_OUT = 64
D_PE = 128

NC = 2
NS = 16
NW = NC * NS
TILE_E = E // NW
CHUNK = 80
NCHUNK = TILE_E // CHUNK
RPT = N // NS
ZROWS = 25


def _make_sc_segsum(D, with_deg):
  mesh = plsc.VectorSubcoreMesh(core_axis_name="c", subcore_axis_name="s")
  out_type = [jax.ShapeDtypeStruct((N, D), jnp.float32),
              jax.ShapeDtypeStruct((N, D), jnp.float32)]
  scratch = [
      pltpu.VMEM((CHUNK,), jnp.int32),
      pltpu.VMEM((CHUNK,), jnp.int32),
      pltpu.VMEM((CHUNK, D), jnp.float32),
      pltpu.VMEM((ZROWS, D), jnp.float32),
      pltpu.VMEM_SHARED((N, D), jnp.float32),
  ]
  if with_deg:
    out_type += [jax.ShapeDtypeStruct((N, 16), jnp.float32),
                 jax.ShapeDtypeStruct((N, 16), jnp.float32)]
    scratch += [
        pltpu.VMEM((CHUNK, 16), jnp.float32),
        pltpu.VMEM((ZROWS, 16), jnp.float32),
        pltpu.VMEM_SHARED((N, 16), jnp.float32),
    ]

  def body(y_hbm, adj_hbm, *refs):
    if with_deg:
      (p0_hbm, p1_hbm, d0_hbm, d1_hbm,
       srcv, dstv, rows, zbuf, acc, onesv, zbuf16, dacc) = refs
    else:
      (p0_hbm, p1_hbm, srcv, dstv, rows, zbuf, acc) = refs
    cid = lax.axis_index("c")
    sid = lax.axis_index("s")
    wid = sid * NC + cid
    base_row = sid * RPT

    zero = jnp.zeros((16,), jnp.float32)

    @pl.loop(0, ZROWS)
    def _(r):
      for j in range(D // 16):
        zbuf[r, pl.ds(j * 16, 16)] = zero

    @pl.loop(0, RPT // ZROWS)
    def _(b):
      pltpu.sync_copy(zbuf, acc.at[pl.ds(base_row + b * ZROWS, ZROWS)])

    if with_deg:
      one = jnp.ones((16,), jnp.float32)

      @pl.loop(0, CHUNK)
      def _(r):
        onesv[r, pl.ds(0, 16)] = one

      @pl.loop(0, ZROWS)
      def _(r):
        zbuf16[r, pl.ds(0, 16)] = zero

      @pl.loop(0, RPT // ZROWS)
      def _(b):
        pltpu.sync_copy(zbuf16, dacc.at[pl.ds(base_row + b * ZROWS, ZROWS)])

    plsc.subcore_barrier()

    ebase = wid * TILE_E

    @pl.loop(0, NCHUNK)
    def _(c):
      off = ebase + c * CHUNK
      pltpu.sync_copy(adj_hbm.at[0, pl.ds(off, CHUNK)], srcv)
      pltpu.sync_copy(adj_hbm.at[1, pl.ds(off, CHUNK)], dstv)
      pltpu.sync_copy(y_hbm.at[srcv], rows)
      pltpu.sync_copy(rows, acc.at[dstv], add=True)
      if with_deg:
        pltpu.sync_copy(onesv, dacc.at[dstv], add=True)

    plsc.subcore_barrier()

    row_slc = pl.ds(base_row, RPT)

    @pl.when(cid == 0)
    def _():
      pltpu.sync_copy(acc.at[row_slc], p0_hbm.at[row_slc])
      if with_deg:
        pltpu.sync_copy(dacc.at[row_slc], d0_hbm.at[row_slc])

    @pl.when(cid == 1)
    def _():
      pltpu.sync_copy(acc.at[row_slc], p1_hbm.at[row_slc])
      if with_deg:
        pltpu.sync_copy(dacc.at[row_slc], d1_hbm.at[row_slc])

  return pl.kernel(body, out_type=tuple(out_type), mesh=mesh,
                   scratch_types=scratch)


_sc_pass0 = _make_sc_segsum(D_HID, True)
_sc_pass1 = _make_sc_segsum(D_HID, False)
_sc_pass2 = _make_sc_segsum(D_OUT, False)


BN = 1000
GRID = N // BN
_F32 = jnp.float32


def _row_spec(d):
  return pl.BlockSpec((BN, d), lambda i: (i, 0))


def _full_spec(r, c):
  return pl.BlockSpec((r, c), lambda i: (0, 0))


def _dot(a, b):
  return jnp.dot(a, b, preferred_element_type=_F32)


def _stage_a(x, emb, wr, wn, wp, b, y_o, root_o):
  xv = x[...]
  y_o[...] = _dot(xv, wn[...])
  root_o[...] = _dot(xv, wr[...]) + _dot(emb[...], wp[...]) + b[...]


def _stage_b(p0, p1, d0, d1, root, emb, wr, wn, wp, b, y_o, root_o, recip_o):
  deg = jnp.maximum(d0[...] + d1[...], 1.0)
  rc = 1.0 / deg
  recip_o[...] = rc
  h = jnp.maximum(root[...] + (p0[...] + p1[...]) * rc[:, :1], 0.0)
  y_o[...] = _dot(h, wn[...])
  root_o[...] = _dot(h, wr[...]) + _dot(emb[...], wp[...]) + b[...]


def _stage_c(p0, p1, recip, root, emb, wr, wn, wp, b, y_o, root_o):
  h = jnp.maximum(root[...] + (p0[...] + p1[...]) * recip[:, :1], 0.0)
  y_o[...] = _dot(h, wn[...])
  root_o[...] = _dot(h, wr[...]) + _dot(emb[...], wp[...]) + b[...]


def _stage_d(p0, p1, recip, root, out_o):
  out_o[...] = root[...] + (p0[...] + p1[...]) * recip[:, :1]


def _tc_stage_a(x, emb, wr, wn, wp, b):
  return pl.pallas_call(
      _stage_a,
      grid=(GRID,),
      in_specs=[_row_spec(D_IN), _row_spec(D_PE),
                _full_spec(D_IN, D_HID), _full_spec(D_IN, D_HID),
                _full_spec(D_PE, D_HID), _full_spec(1, D_HID)],
      out_specs=[_row_spec(D_HID), _row_spec(D_HID)],
      out_shape=[jax.ShapeDtypeStruct((N, D_HID), _F32)] * 2,
  )(x, emb, wr, wn, wp, b)


def _tc_stage_b(p0, p1, d0, d1, root, emb, wr, wn, wp, b):
  return pl.pallas_call(
      _stage_b,
      grid=(GRID,),
      in_specs=[_row_spec(D_HID), _row_spec(D_HID),
                _row_spec(16), _row_spec(16),
                _row_spec(D_HID), _row_spec(D_PE),
                _full_spec(D_HID, D_HID), _full_spec(D_HID, D_HID),
                _full_spec(D_PE, D_HID), _full_spec(1, D_HID)],
      out_specs=[_row_spec(D_HID), _row_spec(D_HID), _row_spec(16)],
      out_shape=[jax.ShapeDtypeStruct((N, D_HID), _F32),
                 jax.ShapeDtypeStruct((N, D_HID), _F32),
                 jax.ShapeDtypeStruct((N, 16), _F32)],
  )(p0, p1, d0, d1, root, emb, wr, wn, wp, b)


def _tc_stage_c(p0, p1, recip, root, emb, wr, wn, wp, b):
  return pl.pallas_call(
      _stage_c,
      grid=(GRID,),
      in_specs=[_row_spec(D_HID), _row_spec(D_HID), _row_spec(16),
                _row_spec(D_HID), _row_spec(D_PE),
                _full_spec(D_HID, D_OUT), _full_spec(D_HID, D_OUT),
                _full_spec(D_PE, D_OUT), _full_spec(1, D_OUT)],
      out_specs=[_row_spec(D_OUT), _row_spec(D_OUT)],
      out_shape=[jax.ShapeDtypeStruct((N, D_OUT), _F32)] * 2,
  )(p0, p1, recip, root, emb, wr, wn, wp, b)


def _tc_stage_d(p0, p1, recip, root):
  return pl.pallas_call(
      _stage_d,
      grid=(GRID,),
      in_specs=[_row_spec(D_OUT), _row_spec(D_OUT), _row_spec(16),
                _row_spec(D_OUT)],
      out_specs=_row_spec(D_OUT),
      out_shape=jax.ShapeDtypeStruct((N, D_OUT), _F32),
  )(p0, p1, recip, root)


def kernel(x, adj_t, embeddings, Wr0, Wn0, Wp0, b0,
           Wr1, Wn1, Wp1, b1, Wr2, Wn2, Wp2, b2):
  b0r = b0.reshape(1, D_HID)
  b1r = b1.reshape(1, D_HID)
  b2r = b2.reshape(1, D_OUT)

  y0, root0 = _tc_stage_a(x, embeddings, Wr0, Wn0, Wp0, b0r)
  p0, p1, d0, d1 = _sc_pass0(y0, adj_t)
  y1, root1, recip = _tc_stage_b(p0, p1, d0, d1, root0, embeddings,
                                 Wr1, Wn1, Wp1, b1r)
  q0, q1 = _sc_pass1(y1, adj_t)
  y2, root2 = _tc_stage_c(q0, q1, recip, root1, embeddings,
                          Wr2, Wn2, Wp2, b2r)
  s0, s1 = _sc_pass2(y2, adj_t)
  return _tc_stage_d(s0, s1, recip, root2)


# speedup vs baseline: 4.1070x; 4.1070x over previous
import functools

import jax
import jax.numpy as jnp
from jax import lax
from jax.experimental import pallas as pl
from jax.experimental.pallas import tpu as pltpu
from jax.experimental.pallas import tpu_sc as plsc

N = 10000
E = 320000
D_IN = 128
D_HID = 128
D_OUT = 64
D_PE = 128

NC = 2
NS = 16
NW = NC * NS
TILE_E = E // NW
CHUNK = 80
NCHUNK = TILE_E // CHUNK
RPT = 624
TAIL_BASE = NS * RPT
TAIL = N - TAIL_BASE
ZROWS = 16


def _zero_acc(zbuf, acc, sid, base_row):
  zero = jnp.zeros((16,), jnp.float32)
  D = zbuf.shape[1]

  @pl.loop(0, ZROWS)
  def _(r):
    for j in range(D // 16):
      zbuf[r, pl.ds(j * 16, 16)] = zero

  @pl.loop(0, RPT // ZROWS)
  def _(b):
    pltpu.sync_copy(zbuf, acc.at[pl.ds(base_row + b * ZROWS, ZROWS)])

  @pl.when(sid == NS - 1)
  def _():
    pltpu.sync_copy(zbuf, acc.at[pl.ds(TAIL_BASE, TAIL)])


def _drain_acc(acc, out_hbm, base_row, sid):
  row_slc = pl.ds(base_row, RPT)
  tail_slc = pl.ds(TAIL_BASE, TAIL)
  pltpu.sync_copy(acc.at[row_slc], out_hbm.at[row_slc])

  @pl.when(sid == NS - 1)
  def _():
    pltpu.sync_copy(acc.at[tail_slc], out_hbm.at[tail_slc])


def _make_sc_segsum():
  mesh = plsc.VectorSubcoreMesh(core_axis_name="c", subcore_axis_name="s")
  out_type = (jax.ShapeDtypeStruct((N, D_HID), jnp.float32),
              jax.ShapeDtypeStruct((N, D_HID), jnp.float32))
  scratch = [
      pltpu.VMEM((CHUNK,), jnp.int32),
      pltpu.VMEM((CHUNK,), jnp.int32),
      pltpu.VMEM((CHUNK, D_HID), jnp.float32),
      pltpu.VMEM((ZROWS, D_HID), jnp.float32),
      pltpu.VMEM_SHARED((N, D_HID), jnp.float32),
      pltpu.SemaphoreType.DMA,
  ]

  def body(y_hbm, src_hbm, dst_hbm, p0_hbm, p1_hbm,
           srcv, dstv, rows, zbuf, acc, sem):
    cid = lax.axis_index("c")
    sid = lax.axis_index("s")
    wid = sid * NC + cid
    base_row = sid * RPT

    _zero_acc(zbuf, acc, sid, base_row)
    plsc.subcore_barrier()

    ebase = wid * TILE_E

    @pl.loop(0, NCHUNK)
    def _(c):
      off = ebase + c * CHUNK
      pltpu.sync_copy(src_hbm.at[pl.ds(off, CHUNK)], srcv)
      pltpu.sync_copy(dst_hbm.at[pl.ds(off, CHUNK)], dstv)
      pltpu.async_copy(y_hbm.at[srcv], rows, sem).wait()
      pltpu.sync_copy(rows, acc.at[dstv], add=True)

    plsc.subcore_barrier()

    @pl.when(cid == 0)
    def _():
      _drain_acc(acc, p0_hbm, base_row, sid)

    @pl.when(cid == 1)
    def _():
      _drain_acc(acc, p1_hbm, base_row, sid)

  return pl.kernel(body, out_type=out_type, mesh=mesh, scratch_types=scratch)


def _make_sc_segsum_deg():
  mesh = plsc.VectorSubcoreMesh(core_axis_name="c", subcore_axis_name="s")
  out_type = (jax.ShapeDtypeStruct((N, D_HID), jnp.float32),
              jax.ShapeDtypeStruct((N, D_HID), jnp.float32))
  scratch = [
      pltpu.VMEM((CHUNK,), jnp.int32),
      pltpu.VMEM((CHUNK,), jnp.int32),
      pltpu.VMEM((CHUNK, D_HID), jnp.float32),
      pltpu.VMEM((ZROWS, D_HID), jnp.float32),
      pltpu.VMEM_SHARED((N, D_HID), jnp.float32),
      pltpu.SemaphoreType.DMA,
  ]
  EPT = E // NS
  NCHUNK0 = EPT // CHUNK

  def body(y_hbm, src_hbm, dst_hbm, sum_hbm, deg_hbm,
           srcv, dstv, rows, zbuf, acc, sem):
    cid = lax.axis_index("c")
    sid = lax.axis_index("s")
    base_row = sid * RPT

    _zero_acc(zbuf, acc, sid, base_row)

    @pl.when(cid == 1)
    def _():
      one = jnp.ones((16,), jnp.float32)

      @pl.loop(0, CHUNK)
      def _(r):
        for j in range(D_HID // 16):
          rows[r, pl.ds(j * 16, 16)] = one

    plsc.subcore_barrier()

    ebase = sid * EPT

    @pl.when(cid == 0)
    def _():
      @pl.loop(0, NCHUNK0)
      def _(c):
        off = ebase + c * CHUNK
        pltpu.sync_copy(src_hbm.at[pl.ds(off, CHUNK)], srcv)
        pltpu.sync_copy(dst_hbm.at[pl.ds(off, CHUNK)], dstv)
        pltpu.async_copy(y_hbm.at[srcv], rows, sem).wait()
        pltpu.sync_copy(rows, acc.at[dstv], add=True)

    @pl.when(cid == 1)
    def _():
      @pl.loop(0, NCHUNK0)
      def _(c):
        off = ebase + c * CHUNK
        pltpu.sync_copy(dst_hbm.at[pl.ds(off, CHUNK)], dstv)
        pltpu.sync_copy(rows, acc.at[dstv], add=True)

    plsc.subcore_barrier()

    @pl.when(cid == 0)
    def _():
      _drain_acc(acc, sum_hbm, base_row, sid)

    @pl.when(cid == 1)
    def _():
      _drain_acc(acc, deg_hbm, base_row, sid)

  return pl.kernel(body, out_type=out_type, mesh=mesh, scratch_types=scratch)


_sc_pass0 = _make_sc_segsum_deg()
_sc_pass1 = _make_sc_segsum()
_sc_pass2 = _make_sc_segsum()


BN = 1000
GRID = N // BN
_F32 = jnp.float32


def _row_spec(d):
  return pl.BlockSpec((BN, d), lambda i: (i, 0))


def _full_spec(r, c):
  return pl.BlockSpec((r, c), lambda i: (0, 0))


def _dot(a, b):
  return jnp.dot(a, b, preferred_element_type=_F32)


def _stage_a(x, emb, wr, wn, wp, b, y_o, root_o):
  xv = x[...]
  y_o[...] = _dot(xv, wn[...])
  root_o[...] = _dot(xv, wr[...]) + _dot(emb[...], wp[...]) + b[...]


def _stage_b(psum, pdeg, root, emb, wr, wn, wp, b, y_o, root_o, recip_o):
  rc = 1.0 / jnp.maximum(pdeg[...], 1.0)
  recip_o[...] = rc[:, :16]
  h = jnp.maximum(root[...] + psum[...] * rc, 0.0)
  y_o[...] = _dot(h, wn[...])
  root_o[...] = _dot(h, wr[...]) + _dot(emb[...], wp[...]) + b[...]


def _stage_c(p0, p1, recip, root, emb, wr, wn, wp, b, y_o, root_o):
  h = jnp.maximum(root[...] + (p0[...] + p1[...]) * recip[:, :1], 0.0)
  y_o[:, :D_OUT] = _dot(h, wn[...])
  y_o[:, D_OUT:] = jnp.zeros((BN, D_HID - D_OUT), _F32)
  root_o[...] = _dot(h, wr[...]) + _dot(emb[...], wp[...]) + b[...]


def _stage_d(p0, p1, recip, root, out_o):
  out_o[...] = root[...] + (p0[:, :D_OUT] + p1[:, :D_OUT]) * recip[:, :1]


def _tc_stage_a(x, emb, wr, wn, wp, b):
  return pl.pallas_call(
      _stage_a,
      grid=(GRID,),
      in_specs=[_row_spec(D_IN), _row_spec(D_PE),
                _full_spec(D_IN, D_HID), _full_spec(D_IN, D_HID),
                _full_spec(D_PE, D_HID), _full_spec(1, D_HID)],
      out_specs=[_row_spec(D_HID), _row_spec(D_HID)],
      out_shape=[jax.ShapeDtypeStruct((N, D_HID), _F32)] * 2,
  )(x, emb, wr, wn, wp, b)


def _tc_stage_b(psum, pdeg, root, emb, wr, wn, wp, b):
  return pl.pallas_call(
      _stage_b,
      grid=(GRID,),
      in_specs=[_row_spec(D_HID), _row_spec(D_HID),
                _row_spec(D_HID), _row_spec(D_PE),
                _full_spec(D_HID, D_HID), _full_spec(D_HID, D_HID),
                _full_spec(D_PE, D_HID), _full_spec(1, D_HID)],
      out_specs=[_row_spec(D_HID), _row_spec(D_HID), _row_spec(16)],
      out_shape=[jax.ShapeDtypeStruct((N, D_HID), _F32),
                 jax.ShapeDtypeStruct((N, D_HID), _F32),
                 jax.ShapeDtypeStruct((N, 16), _F32)],
  )(psum, pdeg, root, emb, wr, wn, wp, b)


def _tc_stage_c(p0, p1, recip, root, emb, wr, wn, wp, b):
  return pl.pallas_call(
      _stage_c,
      grid=(GRID,),
      in_specs=[_row_spec(D_HID), _row_spec(D_HID), _row_spec(16),
                _row_spec(D_HID), _row_spec(D_PE),
                _full_spec(D_HID, D_OUT), _full_spec(D_HID, D_OUT),
                _full_spec(D_PE, D_OUT), _full_spec(1, D_OUT)],
      out_specs=[_row_spec(D_HID), _row_spec(D_OUT)],
      out_shape=[jax.ShapeDtypeStruct((N, D_HID), _F32),
                 jax.ShapeDtypeStruct((N, D_OUT), _F32)],
  )(p0, p1, recip, root, emb, wr, wn, wp, b)


def _tc_stage_d(p0, p1, recip, root):
  return pl.pallas_call(
      _stage_d,
      grid=(GRID,),
      in_specs=[_row_spec(D_HID), _row_spec(D_HID), _row_spec(16),
                _row_spec(D_OUT)],
      out_specs=_row_spec(D_OUT),
      out_shape=jax.ShapeDtypeStruct((N, D_OUT), _F32),
  )(p0, p1, recip, root)


def kernel(x, adj_t, embeddings, Wr0, Wn0, Wp0, b0,
           Wr1, Wn1, Wp1, b1, Wr2, Wn2, Wp2, b2):
  b0r = b0.reshape(1, D_HID)
  b1r = b1.reshape(1, D_HID)
  b2r = b2.reshape(1, D_OUT)

  src = adj_t[0]
  dst = adj_t[1]

  y0, root0 = _tc_stage_a(x, embeddings, Wr0, Wn0, Wp0, b0r)
  psum, pdeg = _sc_pass0(y0, src, dst)
  y1, root1, recip = _tc_stage_b(psum, pdeg, root0, embeddings,
                                 Wr1, Wn1, Wp1, b1r)
  q0, q1 = _sc_pass1(y1, src, dst)
  y2, root2 = _tc_stage_c(q0, q1, recip, root1, embeddings,
                          Wr2, Wn2, Wp2, b2r)
  s0, s1 = _sc_pass2(y2, src, dst)
  return _tc_stage_d(s0, s1, recip, root2)
